# baseline (speedup 1.0000x reference)
import functools

import jax
import jax.numpy as jnp
from jax import lax
from jax.experimental import pallas as pl
from jax.experimental.pallas import tpu as pltpu
from jax.experimental.pallas import tpu_sc as plsc

B = 16384
D = 32
NC, NS = 2, 16
NW = NC * NS
BPW = B // NW
CH = 128
NCH = BPW // CH
NROW = B // CH

_mesh = plsc.VectorSubcoreMesh(core_axis_name="c", subcore_axis_name="s")


@functools.partial(
    pl.kernel,
    out_type=(
        jax.ShapeDtypeStruct((NROW, CH, D), jnp.float32),
        jax.ShapeDtypeStruct((NROW, CH, D), jnp.float32),
    ),
    mesh=_mesh,
    compiler_params=pltpu.CompilerParams(use_tc_tiling_on_sc=False),
    scratch_types=[
        pltpu.VMEM((NCH, CH), jnp.int32),
        pltpu.VMEM((NCH, CH), jnp.int32),
        pltpu.VMEM((NCH, CH, D), jnp.float32),
        pltpu.VMEM((NCH, CH, D), jnp.float32),
        pltpu.SemaphoreType.DMA,
    ],
)
def _sc_gather(uid_hbm, mid_hbm, ut_hbm, mt_hbm, ue_hbm, me_hbm,
               uidx, midx, urows, mrows, sem):
    wid = lax.axis_index("s") * NC + lax.axis_index("c")
    row0 = wid * NCH
    pltpu.sync_copy(uid_hbm.at[pl.ds(row0, NCH)], uidx)
    pltpu.sync_copy(mid_hbm.at[pl.ds(row0, NCH)], midx)
    copies = []
    for j in range(NCH):
        copies.append(pltpu.async_copy(ut_hbm.at[uidx.at[j]], urows.at[j], sem))
        copies.append(pltpu.async_copy(mt_hbm.at[midx.at[j]], mrows.at[j], sem))
    for c in copies:
        c.wait()
    pltpu.sync_copy(urows, ue_hbm.at[pl.ds(row0, NCH)])
    pltpu.sync_copy(mrows, me_hbm.at[pl.ds(row0, NCH)])


BLK = 2048


def _mlp_body(ue, me, w1u, w1m, b1, w2, b2, w3t, b3, out):
    h = jnp.dot(ue[...], w1u[...], preferred_element_type=jnp.float32)
    h = h + jnp.dot(me[...], w1m[...], preferred_element_type=jnp.float32)
    h = jnp.maximum(h + b1[...], 0.0)
    h = jnp.dot(h, w2[...], preferred_element_type=jnp.float32) + b2[...]
    h = jnp.maximum(h, 0.0)
    out[...] = jnp.sum(h * w3t[...], axis=1, keepdims=True) + b3[...]


_mlp = pl.pallas_call(
    _mlp_body,
    grid=(B // BLK,),
    in_specs=[
        pl.BlockSpec((BLK, D), lambda i: (i, 0)),
        pl.BlockSpec((BLK, D), lambda i: (i, 0)),
        pl.BlockSpec((D, 256), lambda i: (0, 0)),
        pl.BlockSpec((D, 256), lambda i: (0, 0)),
        pl.BlockSpec((1, 256), lambda i: (0, 0)),
        pl.BlockSpec((256, 64), lambda i: (0, 0)),
        pl.BlockSpec((1, 64), lambda i: (0, 0)),
        pl.BlockSpec((1, 64), lambda i: (0, 0)),
        pl.BlockSpec((1, 1), lambda i: (0, 0)),
    ],
    out_specs=pl.BlockSpec((BLK, 1), lambda i: (i, 0)),
    out_shape=jax.ShapeDtypeStruct((B, 1), jnp.float32),
)


def kernel(user_id, movie_title, user_table, movie_table, W1, b1, W2, b2, W3, b3):
    uid = user_id.astype(jnp.int32).reshape(NROW, CH)
    mid = movie_title.astype(jnp.int32).reshape(NROW, CH)
    ue, me = _sc_gather(uid, mid, user_table, movie_table)
    return _mlp(
        ue.reshape(B, D),
        me.reshape(B, D),
        W1[:D],
        W1[D:],
        b1.reshape(1, 256),
        W2,
        b2.reshape(1, 64),
        W3.reshape(1, 64),
        b3.reshape(1, 1),
    )
